# baseline (speedup 1.0000x reference)
import functools

import jax
import jax.numpy as jnp
from jax import lax
from jax.experimental import pallas as pl
from jax.experimental.pallas import tpu as pltpu
from jax.experimental.pallas import tpu_sc as plsc

N, E, D, H = 10000, 320000, 128, 256
NC, NS = 2, 16
DC = D // NC
EPT = E // NS
CHUNK = 128
PADE = 20480
NCHUNK = PADE // CHUNK
NPAD = 10240
RPT = NPAD // NS

NBUF = 5
NT = NCHUNK // NBUF


@functools.cache
def _make_sc_agg():
    mesh = plsc.VectorSubcoreMesh(
        core_axis_name="c", subcore_axis_name="s",
        num_cores=NC, num_subcores=NS)

    @functools.partial(
        pl.kernel,
        out_type=jax.ShapeDtypeStruct((NC, NPAD, DC), jnp.float32),
        mesh=mesh,
        scratch_types=[
            pltpu.VMEM((NCHUNK, CHUNK), jnp.int32),
            pltpu.VMEM((NCHUNK, CHUNK), jnp.int32),
            pltpu.VMEM((NBUF, CHUNK, DC), jnp.float32),
            pltpu.VMEM_SHARED((NPAD, DC), jnp.float32),
            [pltpu.SemaphoreType.DMA] * NBUF,
            [pltpu.SemaphoreType.DMA] * NBUF,
            pltpu.SemaphoreType.DMA,
        ],
        compiler_params=pltpu.CompilerParams(use_tc_tiling_on_sc=False),
    )
    def _sc_agg(x_hbm, edge_hbm, zero_hbm, out_hbm,
                sidx, didx, rows, acc, gsems, ssems, zsem):
        c = lax.axis_index("c")
        s = lax.axis_index("s")
        r0 = s * RPT
        zdesc = pltpu.async_copy(
            zero_hbm, acc.at[pl.ds(r0, RPT)], zsem)
        pltpu.sync_copy(edge_hbm.at[0, s], sidx)
        pltpu.sync_copy(edge_hbm.at[1, s], didx)
        zdesc.wait()
        plsc.subcore_barrier()

        def gather(g, b):
            for k in range(CHUNK // 16):
                v = sidx[g, pl.ds(k * 16, 16)]
                sidx[g, pl.ds(k * 16, 16)] = v * 2 + c
            return pltpu.async_copy(x_hbm.at[sidx.at[g]], rows.at[b], gsems[b])

        def scatter(g, b):
            return pltpu.async_copy(
                rows.at[b], acc.at[didx.at[g]], ssems[b], add=True)

        for b in range(NBUF):
            gather(b, b)

        def outer(t, carry):
            for b in range(NBUF):
                g = t * NBUF + b
                pltpu.make_async_copy(
                    x_hbm.at[sidx.at[g]], rows.at[b], gsems[b]).wait()
                scatter(g, b)
            for b in range(NBUF):
                g = t * NBUF + b
                pltpu.make_async_copy(
                    rows.at[b], acc.at[didx.at[g]], ssems[b]).wait()
                gather(g + NBUF, b)
            return carry

        lax.fori_loop(0, NT - 1, outer, 0)
        for b in range(NBUF):
            g = (NT - 1) * NBUF + b
            pltpu.make_async_copy(
                x_hbm.at[sidx.at[g]], rows.at[b], gsems[b]).wait()
            scatter(g, b)
        for b in range(NBUF):
            g = (NT - 1) * NBUF + b
            pltpu.make_async_copy(
                rows.at[b], acc.at[didx.at[g]], ssems[b]).wait()
        plsc.subcore_barrier()
        pltpu.sync_copy(acc.at[pl.ds(r0, RPT)], out_hbm.at[c, pl.ds(r0, RPT)])

    return _sc_agg


BN = 1000


def _mlp_body(eps_ref, x_ref, agg_ref, w1_ref, b1_ref, gamma_ref, beta_ref,
              w2_ref, b2_ref, out_ref, h_scr, stats_scr):
    p = pl.program_id(0)
    i = pl.program_id(1)

    @pl.when(p == 0)
    def _():
        scale = 1.0 + eps_ref[0, 0]
        lo = scale * x_ref[:, :DC] + agg_ref[0]
        hi = scale * x_ref[:, DC:] + agg_ref[1]
        h1 = jnp.dot(lo, w1_ref[:DC], preferred_element_type=jnp.float32)
        h1 = h1 + jnp.dot(hi, w1_ref[DC:], preferred_element_type=jnp.float32)
        h1 = h1 + b1_ref[...]
        h_scr[pl.ds(i * BN, BN), :] = h1

        @pl.when(i == 0)
        def _():
            stats_scr[...] = jnp.zeros_like(stats_scr)

        stats_scr[0:1] += jnp.sum(h1, axis=0, keepdims=True)
        stats_scr[1:2] += jnp.sum(h1 * h1, axis=0, keepdims=True)

    @pl.when(p == 1)
    def _():
        mu = stats_scr[0:1] / N
        var = stats_scr[1:2] / N - mu * mu
        a = gamma_ref[...] * lax.rsqrt(var + 1e-5)
        cshift = beta_ref[...] - mu * a
        nrm = h_scr[pl.ds(i * BN, BN), :] * a + cshift
        act = jnp.where(nrm > 0, nrm, jnp.exp(jnp.minimum(nrm, 0.0)) - 1.0)
        out = jnp.dot(act, w2_ref[...], preferred_element_type=jnp.float32)
        out_ref[...] = out + b2_ref[...]


def kernel(x, edge_index, W1, b1, gamma, beta, W2, b2, eps):
    xv = jnp.reshape(x, (NC * N, DC))
    e3 = jnp.reshape(edge_index, (2, NS, EPT))
    pad = jnp.broadcast_to(
        jnp.array([[[0]], [[NPAD - 2]]], jnp.int32), (2, NS, PADE - EPT))
    edge2 = jnp.reshape(
        jnp.concatenate([e3, pad], axis=2), (2, NS, NCHUNK, CHUNK))
    zeros = jnp.zeros((RPT, DC), jnp.float32)
    agg2 = _make_sc_agg()(xv, edge2, zeros)

    eps2 = jnp.reshape(eps, (1, 1))
    grid = N // BN
    out = pl.pallas_call(
        _mlp_body,
        grid=(2, grid),
        in_specs=[
            pl.BlockSpec(memory_space=pltpu.SMEM),
            pl.BlockSpec((BN, D), lambda p, i: (i * (1 - p), 0)),
            pl.BlockSpec((NC, BN, DC), lambda p, i: (0, i * (1 - p), 0)),
            pl.BlockSpec((D, H), lambda p, i: (0, 0)),
            pl.BlockSpec((1, H), lambda p, i: (0, 0)),
            pl.BlockSpec((1, H), lambda p, i: (0, 0)),
            pl.BlockSpec((1, H), lambda p, i: (0, 0)),
            pl.BlockSpec((H, D), lambda p, i: (0, 0)),
            pl.BlockSpec((1, D), lambda p, i: (0, 0)),
        ],
        out_specs=pl.BlockSpec((BN, D), lambda p, i: (i * p, 0)),
        out_shape=jax.ShapeDtypeStruct((N, D), jnp.float32),
        scratch_shapes=[
            pltpu.VMEM((N, H), jnp.float32),
            pltpu.VMEM((2, H), jnp.float32),
        ],
    )(eps2, x, agg2, W1, jnp.reshape(b1, (1, H)),
      jnp.reshape(gamma, (1, H)), jnp.reshape(beta, (1, H)),
      W2, jnp.reshape(b2, (1, D)))
    return out


# speedup vs baseline: 2.9048x; 2.9048x over previous
import functools

import jax
import jax.numpy as jnp
from jax import lax
from jax.experimental import pallas as pl
from jax.experimental.pallas import tpu as pltpu
from jax.experimental.pallas import tpu_sc as plsc

N, E, D, H = 10000, 320000, 128, 256
NC, NS = 2, 16
DC = D // NC
EPT = E // NS
CHUNK = 80
NCHUNK = EPT // CHUNK
NPAD = 10240
RPT = NPAD // NS

NBUF = 5
NT = NCHUNK // NBUF


@functools.cache
def _make_sc_agg():
    mesh = plsc.VectorSubcoreMesh(
        core_axis_name="c", subcore_axis_name="s",
        num_cores=NC, num_subcores=NS)

    @functools.partial(
        pl.kernel,
        out_type=jax.ShapeDtypeStruct((NC, NPAD, DC), jnp.float32),
        mesh=mesh,
        scratch_types=[
            pltpu.VMEM((NCHUNK, CHUNK), jnp.int32),
            pltpu.VMEM((NCHUNK, CHUNK), jnp.int32),
            pltpu.VMEM((NBUF, CHUNK, DC), jnp.float32),
            pltpu.VMEM_SHARED((NPAD, DC), jnp.float32),
            [pltpu.SemaphoreType.DMA] * NBUF,
            [pltpu.SemaphoreType.DMA] * NBUF,
            pltpu.SemaphoreType.DMA,
        ],
        compiler_params=pltpu.CompilerParams(use_tc_tiling_on_sc=False),
    )
    def _sc_agg(x_hbm, edge_hbm, zero_hbm, out_hbm,
                sidx, didx, rows, acc, gsems, ssems, zsem):
        c = lax.axis_index("c")
        s = lax.axis_index("s")
        r0 = s * RPT
        zdesc = pltpu.async_copy(
            zero_hbm, acc.at[pl.ds(r0, RPT)], zsem)
        pltpu.sync_copy(edge_hbm.at[0, s], sidx)
        pltpu.sync_copy(edge_hbm.at[1, s], didx)
        zdesc.wait()
        plsc.subcore_barrier()

        def gather(g, b):
            for k in range(CHUNK // 16):
                v = sidx[g, pl.ds(k * 16, 16)]
                sidx[g, pl.ds(k * 16, 16)] = v * 2 + c
            return pltpu.async_copy(x_hbm.at[sidx.at[g]], rows.at[b], gsems[b])

        def scatter(g, b):
            return pltpu.async_copy(
                rows.at[b], acc.at[didx.at[g]], ssems[b], add=True)

        for b in range(NBUF):
            gather(b, b)

        def outer(t, carry):
            for b in range(NBUF):
                g = t * NBUF + b
                pltpu.make_async_copy(
                    x_hbm.at[sidx.at[g]], rows.at[b], gsems[b]).wait()
                scatter(g, b)
            for b in range(NBUF):
                g = t * NBUF + b
                pltpu.make_async_copy(
                    rows.at[b], acc.at[didx.at[g]], ssems[b]).wait()
                gather(g + NBUF, b)
            return carry

        lax.fori_loop(0, NT - 1, outer, 0)
        for b in range(NBUF):
            g = (NT - 1) * NBUF + b
            pltpu.make_async_copy(
                x_hbm.at[sidx.at[g]], rows.at[b], gsems[b]).wait()
            scatter(g, b)
        for b in range(NBUF):
            g = (NT - 1) * NBUF + b
            pltpu.make_async_copy(
                rows.at[b], acc.at[didx.at[g]], ssems[b]).wait()
        plsc.subcore_barrier()
        pltpu.sync_copy(acc.at[pl.ds(r0, RPT)], out_hbm.at[c, pl.ds(r0, RPT)])

    return _sc_agg


BN = 2000


def _mlp_body(eps_ref, x_ref, agg_ref, w1_ref, b1_ref, gamma_ref, beta_ref,
              w2_ref, b2_ref, out_ref, h_scr, stats_scr):
    p = pl.program_id(0)
    i = pl.program_id(1)

    @pl.when(p == 0)
    def _():
        scale = 1.0 + eps_ref[0, 0]
        lo = scale * x_ref[:, :DC] + agg_ref[0]
        hi = scale * x_ref[:, DC:] + agg_ref[1]
        h1 = jnp.dot(lo, w1_ref[:DC], preferred_element_type=jnp.float32)
        h1 = h1 + jnp.dot(hi, w1_ref[DC:], preferred_element_type=jnp.float32)
        h1 = h1 + b1_ref[...]
        h_scr[pl.ds(i * BN, BN), :] = h1

        @pl.when(i == 0)
        def _():
            stats_scr[...] = jnp.zeros_like(stats_scr)

        stats_scr[0:1] += jnp.sum(h1, axis=0, keepdims=True)
        stats_scr[1:2] += jnp.sum(h1 * h1, axis=0, keepdims=True)

    @pl.when(p == 1)
    def _():
        mu = stats_scr[0:1] / N
        var = stats_scr[1:2] / N - mu * mu
        a = gamma_ref[...] * lax.rsqrt(var + 1e-5)
        cshift = beta_ref[...] - mu * a
        nrm = h_scr[pl.ds(i * BN, BN), :] * a + cshift
        act = jnp.where(nrm > 0, nrm, jnp.exp(jnp.minimum(nrm, 0.0)) - 1.0)
        out = jnp.dot(act, w2_ref[...], preferred_element_type=jnp.float32)
        out_ref[...] = out + b2_ref[...]


def kernel(x, edge_index, W1, b1, gamma, beta, W2, b2, eps):
    xv = jnp.reshape(x, (NC * N, DC))
    edge2 = jnp.reshape(edge_index, (2, NS, NCHUNK, CHUNK))
    zeros = jnp.zeros((RPT, DC), jnp.float32)
    agg2 = _make_sc_agg()(xv, edge2, zeros)

    eps2 = jnp.reshape(eps, (1, 1))
    grid = N // BN
    out = pl.pallas_call(
        _mlp_body,
        grid=(2, grid),
        in_specs=[
            pl.BlockSpec(memory_space=pltpu.SMEM),
            pl.BlockSpec((BN, D), lambda p, i: (i * (1 - p), 0)),
            pl.BlockSpec((NC, BN, DC), lambda p, i: (0, i * (1 - p), 0)),
            pl.BlockSpec((D, H), lambda p, i: (0, 0)),
            pl.BlockSpec((1, H), lambda p, i: (0, 0)),
            pl.BlockSpec((1, H), lambda p, i: (0, 0)),
            pl.BlockSpec((1, H), lambda p, i: (0, 0)),
            pl.BlockSpec((H, D), lambda p, i: (0, 0)),
            pl.BlockSpec((1, D), lambda p, i: (0, 0)),
        ],
        out_specs=pl.BlockSpec((BN, D), lambda p, i: (i * p, 0)),
        out_shape=jax.ShapeDtypeStruct((N, D), jnp.float32),
        scratch_shapes=[
            pltpu.VMEM((N, H), jnp.float32),
            pltpu.VMEM((2, H), jnp.float32),
        ],
    )(eps2, x, agg2, W1, jnp.reshape(b1, (1, H)),
      jnp.reshape(gamma, (1, H)), jnp.reshape(beta, (1, H)),
      W2, jnp.reshape(b2, (1, D)))
    return out


# speedup vs baseline: 2.9229x; 1.0062x over previous
import functools

import jax
import jax.numpy as jnp
from jax import lax
from jax.experimental import pallas as pl
from jax.experimental.pallas import tpu as pltpu
from jax.experimental.pallas import tpu_sc as plsc

N, E, D, H = 10000, 320000, 128, 256
NC, NS = 2, 16
DC = D // NC
EPT = E // NS
CHUNK = 80
NCHUNK = EPT // CHUNK
NPAD = 10240
RPT = NPAD // NS

NBUF = 5
NT = NCHUNK // NBUF


@functools.cache
def _make_sc_agg():
    mesh = plsc.VectorSubcoreMesh(
        core_axis_name="c", subcore_axis_name="s",
        num_cores=NC, num_subcores=NS)

    @functools.partial(
        pl.kernel,
        out_type=jax.ShapeDtypeStruct((NC, NPAD, DC), jnp.float32),
        mesh=mesh,
        scratch_types=[
            pltpu.VMEM((NCHUNK, CHUNK), jnp.int32),
            pltpu.VMEM((NCHUNK, CHUNK), jnp.int32),
            pltpu.VMEM((NBUF, CHUNK, DC), jnp.float32),
            pltpu.VMEM_SHARED((NPAD, DC), jnp.float32),
            [pltpu.SemaphoreType.DMA] * NBUF,
            [pltpu.SemaphoreType.DMA] * NBUF,
            pltpu.SemaphoreType.DMA,
        ],
        compiler_params=pltpu.CompilerParams(use_tc_tiling_on_sc=False),
    )
    def _sc_agg(x_hbm, edge_hbm, zero_hbm, out_hbm,
                sidx, didx, rows, acc, gsems, ssems, zsem):
        c = lax.axis_index("c")
        s = lax.axis_index("s")
        r0 = s * RPT
        zdesc = pltpu.async_copy(
            zero_hbm, acc.at[pl.ds(r0, RPT)], zsem)
        pltpu.sync_copy(edge_hbm.at[0, s], sidx)
        pltpu.sync_copy(edge_hbm.at[1, s], didx)
        zdesc.wait()
        plsc.subcore_barrier()

        def gather(g, b):
            for k in range(CHUNK // 16):
                v = sidx[g, pl.ds(k * 16, 16)]
                sidx[g, pl.ds(k * 16, 16)] = v * 2 + c
            return pltpu.async_copy(x_hbm.at[sidx.at[g]], rows.at[b], gsems[b])

        def scatter(g, b):
            return pltpu.async_copy(
                rows.at[b], acc.at[didx.at[g]], ssems[b], add=True)

        for b in range(NBUF):
            gather(b, b)

        def outer(t, carry):
            for b in range(NBUF):
                g = t * NBUF + b
                pltpu.make_async_copy(
                    x_hbm.at[sidx.at[g]], rows.at[b], gsems[b]).wait()
                scatter(g, b)
            for b in range(NBUF):
                g = t * NBUF + b
                pltpu.make_async_copy(
                    rows.at[b], acc.at[didx.at[g]], ssems[b]).wait()
                gather(g + NBUF, b)
            return carry

        lax.fori_loop(0, NT - 1, outer, 0)
        for b in range(NBUF):
            g = (NT - 1) * NBUF + b
            pltpu.make_async_copy(
                x_hbm.at[sidx.at[g]], rows.at[b], gsems[b]).wait()
            scatter(g, b)
        for b in range(NBUF):
            g = (NT - 1) * NBUF + b
            pltpu.make_async_copy(
                rows.at[b], acc.at[didx.at[g]], ssems[b]).wait()
        plsc.subcore_barrier()
        pltpu.sync_copy(acc.at[pl.ds(r0, RPT)], out_hbm.at[c, pl.ds(r0, RPT)])

    return _sc_agg


BN = 5000


def _mlp_body(eps_ref, x_ref, agg_ref, w1_ref, b1_ref, gamma_ref, beta_ref,
              w2_ref, b2_ref, out_ref, h_scr, stats_scr):
    p = pl.program_id(0)
    i = pl.program_id(1)

    @pl.when(p == 0)
    def _():
        scale = 1.0 + eps_ref[0, 0]
        lo = scale * x_ref[:, :DC] + agg_ref[0]
        hi = scale * x_ref[:, DC:] + agg_ref[1]
        h1 = jnp.dot(lo, w1_ref[:DC], preferred_element_type=jnp.float32)
        h1 = h1 + jnp.dot(hi, w1_ref[DC:], preferred_element_type=jnp.float32)
        h1 = h1 + b1_ref[...]
        h_scr[pl.ds(i * BN, BN), :] = h1

        @pl.when(i == 0)
        def _():
            stats_scr[...] = jnp.zeros_like(stats_scr)

        stats_scr[0:1] += jnp.sum(h1, axis=0, keepdims=True)
        stats_scr[1:2] += jnp.sum(h1 * h1, axis=0, keepdims=True)

    @pl.when(p == 1)
    def _():
        mu = stats_scr[0:1] / N
        var = stats_scr[1:2] / N - mu * mu
        a = gamma_ref[...] * lax.rsqrt(var + 1e-5)
        cshift = beta_ref[...] - mu * a
        nrm = h_scr[pl.ds(i * BN, BN), :] * a + cshift
        act = jnp.where(nrm > 0, nrm, jnp.exp(jnp.minimum(nrm, 0.0)) - 1.0)
        out = jnp.dot(act, w2_ref[...], preferred_element_type=jnp.float32)
        out_ref[...] = out + b2_ref[...]


def kernel(x, edge_index, W1, b1, gamma, beta, W2, b2, eps):
    xv = jnp.reshape(x, (NC * N, DC))
    edge2 = jnp.reshape(edge_index, (2, NS, NCHUNK, CHUNK))
    zeros = jnp.zeros((RPT, DC), jnp.float32)
    agg2 = _make_sc_agg()(xv, edge2, zeros)

    eps2 = jnp.reshape(eps, (1, 1))
    grid = N // BN
    out = pl.pallas_call(
        _mlp_body,
        grid=(2, grid),
        in_specs=[
            pl.BlockSpec(memory_space=pltpu.SMEM),
            pl.BlockSpec((BN, D), lambda p, i: (i * (1 - p), 0)),
            pl.BlockSpec((NC, BN, DC), lambda p, i: (0, i * (1 - p), 0)),
            pl.BlockSpec((D, H), lambda p, i: (0, 0)),
            pl.BlockSpec((1, H), lambda p, i: (0, 0)),
            pl.BlockSpec((1, H), lambda p, i: (0, 0)),
            pl.BlockSpec((1, H), lambda p, i: (0, 0)),
            pl.BlockSpec((H, D), lambda p, i: (0, 0)),
            pl.BlockSpec((1, D), lambda p, i: (0, 0)),
        ],
        out_specs=pl.BlockSpec((BN, D), lambda p, i: (i * p, 0)),
        out_shape=jax.ShapeDtypeStruct((N, D), jnp.float32),
        scratch_shapes=[
            pltpu.VMEM((N, H), jnp.float32),
            pltpu.VMEM((2, H), jnp.float32),
        ],
    )(eps2, x, agg2, W1, jnp.reshape(b1, (1, H)),
      jnp.reshape(gamma, (1, H)), jnp.reshape(beta, (1, H)),
      W2, jnp.reshape(b2, (1, D)))
    return out
